# baseline (speedup 1.0000x reference)
import functools

import jax
import jax.numpy as jnp
from jax import lax
from jax.experimental import pallas as pl
from jax.experimental.pallas import tpu as pltpu
from jax.experimental.pallas import tpu_sc as plsc

N_NODES = 10000
E_EDGES = 320000
DIM = 128
D_EDGE = 16
K = 32
NCORE = 2
NSUB = 16
NW = NCORE * NSUB
NB = E_EDGES // K
N_PAD = 10240
ROWS_PT = N_PAD // NSUB
DEN_ROWS = N_PAD // DIM

_SCALE = 1.0 / (DIM ** 0.5)


def _tables_body(f_ref, w_ref, b_ref, sf_ref, d_ref):
    f = f_ref[...]
    h = jnp.dot(f, w_ref[...], preferred_element_type=jnp.float32)
    h = h + b_ref[...]
    d_ref[...] = h
    col = lax.broadcasted_iota(jnp.int32, h.shape, 1)
    sf_ref[:, :DIM] = jnp.where(col < D_EDGE, h, jnp.tanh(h))
    sf_ref[:, DIM:] = f


def _make_tables(features, w, b):
    blk = 1000
    grid = N_NODES // blk
    return pl.pallas_call(
        _tables_body,
        grid=(grid,),
        in_specs=[
            pl.BlockSpec((blk, DIM), lambda i: (i, 0)),
            pl.BlockSpec((DIM, DIM), lambda i: (0, 0)),
            pl.BlockSpec((1, DIM), lambda i: (0, 0)),
        ],
        out_specs=[
            pl.BlockSpec((blk, 2 * DIM), lambda i: (i, 0)),
            pl.BlockSpec((blk, DIM), lambda i: (i, 0)),
        ],
        out_shape=[
            jax.ShapeDtypeStruct((N_NODES, 2 * DIM), jnp.float32),
            jax.ShapeDtypeStruct((N_NODES, DIM), jnp.float32),
        ],
    )(features, w, b)


def _edge_body(idx_hbm, sf_hbm, d_hbm, zero_hbm,
               out_hbm, den_hbm,
               cbuf_3, srcbuf, dst2,
               sf2_rows, d2_rows, mrow,
               den_tile, accum, sem, sem_pre, sem_sc):
    c = lax.axis_index("c")
    sub = lax.axis_index("s")
    w = c * NSUB + sub
    r0 = sub * ROWS_PT
    iota16 = lax.iota(jnp.int32, 16)

    pltpu.sync_copy(zero_hbm.at[pl.ds(r0, ROWS_PT), :],
                    accum.at[pl.ds(r0, ROWS_PT), :])
    zero16 = jnp.zeros((16,), jnp.float32)

    def zden(i, c2):
        for j in range(8):
            den_tile[i, pl.ds(16 * j, 16)] = zero16
        return c2

    lax.fori_loop(0, DEN_ROWS, zden, 0)
    plsc.subcore_barrier()

    scale = jnp.float32(_SCALE)
    nb_w = NB // NW + jnp.where(w < NB % NW, 1, 0)

    CROWS = 1 + K * D_EDGE // 128
    cmax = CROWS * NB - 1
    ROWB = K * DIM * 4

    pltpu.async_copy(idx_hbm.at[jnp.minimum(CROWS * w + iota16, cmax)],
                     cbuf_3.at[pl.ds(0, 16), :], sem_pre)

    def step(r, carry):
        @pl.when(r < nb_w)
        def _issue():
            t = w + r * NW
            sl = (r & 1) * K
            csl = lax.rem(r, 3) * 16
            pltpu.make_async_copy(
                idx_hbm.at[jnp.minimum(CROWS * t + iota16, cmax)],
                cbuf_3.at[pl.ds(csl, 16), :], sem_pre).wait()

            @pl.when(r >= 2)
            def _wait_sc():
                def dsc(i2, c3):
                    pltpu.make_async_copy(
                        idx_hbm.at[jnp.minimum(iota16, cmax)],
                        cbuf_3.at[pl.ds(0, 16), :], sem_sc).wait()
                    return c3

                lax.fori_loop(0, K * DIM * 4 // 8192, dsc, 0)
            for g in range(K // 16):
                srcbuf[pl.ds(16 * g, 16)] = (
                    cbuf_3[csl, pl.ds(16 * g, 16)].astype(jnp.int32))
                dst2[r & 1, pl.ds(16 * g, 16)] = (
                    cbuf_3[csl, pl.ds(K + 16 * g, 16)].astype(jnp.int32))
            pltpu.async_copy(sf_hbm.at[srcbuf],
                             sf2_rows.at[pl.ds(sl, K), :], sem)
            pltpu.async_copy(d_hbm.at[dst2.at[r & 1]],
                             d2_rows.at[pl.ds(sl, K), :], sem)
            tn = t + NW
            nsl = lax.rem(r + 1, 3) * 16
            pltpu.async_copy(idx_hbm.at[jnp.minimum(CROWS * tn + iota16, cmax)],
                             cbuf_3.at[pl.ds(nsl, 16), :], sem_pre)

        @pl.when(r > 0)
        def _compute():
            b = r - 1
            sl = (b & 1) * K
            csl = lax.rem(b, 3) * 16
            def drain1(i2, c3):
                pltpu.make_async_copy(
                    idx_hbm.at[jnp.minimum(iota16, cmax)],
                    cbuf_3.at[pl.ds(0, 16), :], sem).wait()
                return c3

            lax.fori_loop(0, 3 * K * DIM * 4 // 8192, drain1, 0)

            def edge_grp(g, c2):
                dstg = dst2[b & 1, pl.ds(g * 16, 16)]
                for i in range(16):
                    e = g * 16 + i
                    s0 = sf2_rows[sl + e, pl.ds(0, 16)]
                    d0 = d2_rows[sl + e, pl.ds(0, 16)]
                    er = cbuf_3[csl + 1 + e // 8, pl.ds((e % 8) * 16, 16)]
                    x2 = jnp.exp((s0 + er) * 2.0)
                    acc = d0 * (1.0 - 2.0 / (x2 + 1.0))
                    for j in range(1, 8):
                        acc = acc + (d2_rows[sl + e, pl.ds(16 * j, 16)]
                                     * sf2_rows[sl + e, pl.ds(16 * j, 16)])
                    for sh in (8, 4, 2, 1):
                        acc = acc + jnp.take(acc, iota16 ^ sh)
                    pv = acc * scale
                    pv = jnp.where(pv >= 0.0, pv, 0.2 * pv)
                    pv = jnp.exp(pv)
                    for j in range(8):
                        mrow[e, pl.ds(16 * j, 16)] = (
                            sf2_rows[sl + e, pl.ds(DIM + 16 * j, 16)] * pv)
                    d_e = dstg[i]
                    a16 = (d_e >> 4) << 4
                    row = a16 >> 7
                    col = a16 & 127
                    win = den_tile[row, pl.ds(col, 16)]
                    hit = iota16 == (d_e - a16)
                    den_tile[row, pl.ds(col, 16)] = (
                        win + jnp.where(hit, pv, 0.0))
                return c2

            lax.fori_loop(0, K // 16, edge_grp, 0)
            pltpu.async_copy(mrow, accum.at[dst2.at[b & 1]], sem_sc,
                             add=True)

        return carry

    lax.fori_loop(0, nb_w + 1, step, 0)

    def dsc_tail(i2, c3):
        pltpu.make_async_copy(idx_hbm.at[jnp.minimum(iota16, cmax)],
                              cbuf_3.at[pl.ds(0, 16), :], sem_sc).wait()
        return c3

    lax.fori_loop(0, 2 * K * DIM * 4 // 8192, dsc_tail, 0)
    fsl = lax.rem(nb_w, 3) * 16
    pltpu.make_async_copy(
        idx_hbm.at[jnp.minimum(CROWS * (w + nb_w * NW) + iota16, cmax)],
        cbuf_3.at[pl.ds(fsl, 16), :], sem_pre).wait()

    plsc.subcore_barrier()
    pltpu.sync_copy(accum.at[pl.ds(r0, ROWS_PT), :],
                    out_hbm.at[c, pl.ds(r0, ROWS_PT), :])

    def den_wb(ch, c2):
        pltpu.async_copy(den_tile.at[pl.ds(ch * 16, 16), :],
                         den_hbm.at[w * DEN_ROWS + ch * 16 + iota16],
                         sem).wait()
        return c2

    lax.fori_loop(0, DEN_ROWS // 16, den_wb, 0)


_edge_pass = functools.partial(
    pl.kernel,
    out_type=[
        jax.ShapeDtypeStruct((NCORE, N_PAD, DIM), jnp.float32),
        jax.ShapeDtypeStruct((NW * DEN_ROWS, DIM), jnp.float32),
    ],
    mesh=plsc.VectorSubcoreMesh(core_axis_name="c", subcore_axis_name="s"),
    scratch_types=[
        pltpu.VMEM((48, 128), jnp.float32),
        pltpu.VMEM((K,), jnp.int32),
        pltpu.VMEM((2, K), jnp.int32),
        pltpu.VMEM((2 * K, 2 * DIM), jnp.float32),
        pltpu.VMEM((2 * K, DIM), jnp.float32),
        pltpu.VMEM((K, DIM), jnp.float32),
        pltpu.VMEM((DEN_ROWS, DIM), jnp.float32),
        pltpu.VMEM_SHARED((N_PAD, DIM), jnp.float32),
        pltpu.SemaphoreType.DMA,
        pltpu.SemaphoreType.DMA,
        pltpu.SemaphoreType.DMA,
    ],
)(_edge_body)


def _den_body(d_ref, o_ref):
    o_ref[...] = jnp.sum(d_ref[...], axis=0)[:, None]


def _den_reduce(denp):
    blk = 1024
    grid = N_PAD // blk
    return pl.pallas_call(
        _den_body,
        grid=(grid,),
        in_specs=[pl.BlockSpec((NW, blk), lambda i: (0, i))],
        out_specs=pl.BlockSpec((blk, 1), lambda i: (i, 0)),
        out_shape=jax.ShapeDtypeStruct((N_PAD, 1), jnp.float32),
    )(denp)


def _final_body(f_ref, acc_ref, den_ref, w1_ref, b1_ref, w2_ref, b2_ref,
                delta_ref, o_ref):
    num = acc_ref[0] + acc_ref[1]
    den = den_ref[...]
    h_neigh = num / (den + 1e-9)
    f = f_ref[...]
    o = (jnp.dot(f + h_neigh, w1_ref[...], preferred_element_type=jnp.float32)
         + jnp.dot(f * h_neigh, w2_ref[...], preferred_element_type=jnp.float32)
         + b1_ref[...] + b2_ref[...])
    o = jnp.where(o >= 0.0, o, 0.2 * o)
    o_ref[...] = o + delta_ref[0, 0]


def _final(features, acc, den, w1, b1, w2, b2, delta):
    blk = 1000
    grid = N_NODES // blk
    return pl.pallas_call(
        _final_body,
        grid=(grid,),
        in_specs=[
            pl.BlockSpec((blk, DIM), lambda i: (i, 0)),
            pl.BlockSpec((NCORE, blk, DIM), lambda i: (0, i, 0)),
            pl.BlockSpec((blk, 1), lambda i: (i, 0)),
            pl.BlockSpec((DIM, DIM), lambda i: (0, 0)),
            pl.BlockSpec((1, DIM), lambda i: (0, 0)),
            pl.BlockSpec((DIM, DIM), lambda i: (0, 0)),
            pl.BlockSpec((1, DIM), lambda i: (0, 0)),
            pl.BlockSpec(memory_space=pltpu.SMEM),
        ],
        out_specs=pl.BlockSpec((blk, DIM), lambda i: (i, 0)),
        out_shape=jax.ShapeDtypeStruct((N_NODES, DIM), jnp.float32),
    )(features, acc, den, w1, b1, w2, b2, delta)


def kernel(indices, features, e_r, num_nodes, W1_w, W1_b, W2_w, W2_b,
           Watt_w, Watt_b):
    f32 = jnp.float32
    src = indices[0].astype(jnp.int32)
    dst = indices[1].astype(jnp.int32)
    features = features.astype(f32)
    e_r = e_r.astype(f32)

    s_tab, d_tab = _make_tables(features, Watt_w.astype(f32),
                                Watt_b.astype(f32).reshape(1, DIM))
    zeros = jnp.zeros((N_PAD, DIM), f32)
    idx_row = jnp.concatenate(
        [src.reshape(NB, K), dst.reshape(NB, K),
         jnp.zeros((NB, 128 - 2 * K), jnp.int32)], axis=1).astype(f32)
    er_rows = e_r.reshape(NB, K * D_EDGE // 128, 128)
    comb = jnp.concatenate([idx_row[:, None, :], er_rows],
                           axis=1).reshape(-1, 128)
    acc, denp = _edge_pass(comb, s_tab, d_tab, zeros)
    den = _den_reduce(denp.reshape(NW, N_PAD))

    delta = jnp.asarray(num_nodes - features.shape[0], f32).reshape(1, 1)
    return _final(features, acc, den, W1_w.astype(f32),
                  W1_b.astype(f32).reshape(1, DIM), W2_w.astype(f32),
                  W2_b.astype(f32).reshape(1, DIM), delta)


# speedup vs baseline: 1.2943x; 1.2943x over previous
import functools

import jax
import jax.numpy as jnp
from jax import lax
from jax.experimental import pallas as pl
from jax.experimental.pallas import tpu as pltpu
from jax.experimental.pallas import tpu_sc as plsc

N_NODES = 10000
E_EDGES = 320000
DIM = 128
D_EDGE = 16
K = 32
NCORE = 2
NSUB = 16
NW = NCORE * NSUB
NB = E_EDGES // K
N_PAD = 10240
ROWS_PT = N_PAD // NSUB
DEN_ROWS = N_PAD // DIM

_SCALE = 1.0 / (DIM ** 0.5)


def _tables_body(f_ref, w_ref, b_ref, s_ref, d_ref):
    h = jnp.dot(f_ref[...], w_ref[...], preferred_element_type=jnp.float32)
    h = h + b_ref[...]
    d_ref[...] = h
    col = lax.broadcasted_iota(jnp.int32, h.shape, 1)
    s_ref[...] = jnp.where(col < D_EDGE, h, jnp.tanh(h))


def _make_tables(features, w, b):
    blk = 1000
    grid = N_NODES // blk
    return pl.pallas_call(
        _tables_body,
        grid=(grid,),
        in_specs=[
            pl.BlockSpec((blk, DIM), lambda i: (i, 0)),
            pl.BlockSpec((DIM, DIM), lambda i: (0, 0)),
            pl.BlockSpec((1, DIM), lambda i: (0, 0)),
        ],
        out_specs=[
            pl.BlockSpec((blk, DIM), lambda i: (i, 0)),
            pl.BlockSpec((blk, DIM), lambda i: (i, 0)),
        ],
        out_shape=[
            jax.ShapeDtypeStruct((N_NODES, DIM), jnp.float32),
            jax.ShapeDtypeStruct((N_NODES, DIM), jnp.float32),
        ],
    )(features, w, b)


def _edge_body(idx_hbm, s_hbm, d_hbm, f_hbm, zero_hbm, dumz_hbm,
               out_hbm, den_hbm,
               cbuf_3, srcbuf, dst2,
               s2_rows, d2_rows, f2_rows,
               dumv, den_tile, accum, sem, sem_pre, sem_sc):
    c = lax.axis_index("c")
    sub = lax.axis_index("s")
    w = c * NSUB + sub
    r0 = sub * ROWS_PT
    iota16 = lax.iota(jnp.int32, 16)

    pltpu.sync_copy(zero_hbm.at[pl.ds(r0, ROWS_PT), :],
                    accum.at[pl.ds(r0, ROWS_PT), :])
    zero16 = jnp.zeros((16,), jnp.float32)

    def zden(i, c2):
        for j in range(8):
            den_tile[i, pl.ds(16 * j, 16)] = zero16
        return c2

    lax.fori_loop(0, DEN_ROWS, zden, 0)
    plsc.subcore_barrier()

    scale = jnp.float32(_SCALE)
    nb_w = NB // NW + jnp.where(w < NB % NW, 1, 0)

    CROWS = 1 + K * D_EDGE // 128
    cmax = CROWS * NB - 1
    ROWB = K * DIM * 4

    pltpu.async_copy(idx_hbm.at[jnp.minimum(CROWS * w + iota16, cmax)],
                     cbuf_3.at[pl.ds(0, 16), :], sem_pre)

    def step(r, carry):
        @pl.when(r < nb_w)
        def _issue():
            t = w + r * NW
            sl = (r & 1) * K
            csl = lax.rem(r, 3) * 16
            pltpu.make_async_copy(
                idx_hbm.at[jnp.minimum(CROWS * t + iota16, cmax)],
                cbuf_3.at[pl.ds(csl, 16), :], sem_pre).wait()

            @pl.when(r >= 2)
            def _wait_sc():
                pltpu.make_async_copy(dumz_hbm, dumv, sem_sc).wait()
            for g in range(K // 16):
                srcbuf[pl.ds(16 * g, 16)] = (
                    cbuf_3[csl, pl.ds(16 * g, 16)].astype(jnp.int32))
                dst2[r & 1, pl.ds(16 * g, 16)] = (
                    cbuf_3[csl, pl.ds(K + 16 * g, 16)].astype(jnp.int32))
            pltpu.async_copy(s_hbm.at[srcbuf],
                             s2_rows.at[pl.ds(sl, K), :], sem)
            pltpu.async_copy(d_hbm.at[dst2.at[r & 1]],
                             d2_rows.at[pl.ds(sl, K), :], sem)
            pltpu.async_copy(f_hbm.at[srcbuf],
                             f2_rows.at[pl.ds(sl, K), :], sem)
            tn = t + NW
            nsl = lax.rem(r + 1, 3) * 16
            pltpu.async_copy(idx_hbm.at[jnp.minimum(CROWS * tn + iota16, cmax)],
                             cbuf_3.at[pl.ds(nsl, 16), :], sem_pre)

        @pl.when(r > 0)
        def _compute():
            b = r - 1
            sl = (b & 1) * K
            csl = lax.rem(b, 3) * 16
            def drain1(i2, c3):
                pltpu.make_async_copy(dumz_hbm, dumv, sem).wait()
                return c3

            lax.fori_loop(0, 3 * K * DIM * 4 // 16384, drain1, 0)

            def edge_grp(g, c2):
                dstg = dst2[b & 1, pl.ds(g * 16, 16)]
                for i in range(16):
                    e = g * 16 + i
                    s0 = s2_rows[sl + e, pl.ds(0, 16)]
                    d0 = d2_rows[sl + e, pl.ds(0, 16)]
                    er = cbuf_3[csl + 1 + e // 8, pl.ds((e % 8) * 16, 16)]
                    x2 = jnp.exp((s0 + er) * 2.0)
                    acc = d0 * (1.0 - 2.0 / (x2 + 1.0))
                    for j in range(1, 8):
                        acc = acc + (d2_rows[sl + e, pl.ds(16 * j, 16)]
                                     * s2_rows[sl + e, pl.ds(16 * j, 16)])
                    for sh in (8, 4, 2, 1):
                        acc = acc + jnp.take(acc, iota16 ^ sh)
                    pv = acc * scale
                    pv = jnp.where(pv >= 0.0, pv, 0.2 * pv)
                    pv = jnp.exp(pv)
                    for j in range(8):
                        f2_rows[sl + e, pl.ds(16 * j, 16)] = (
                            f2_rows[sl + e, pl.ds(16 * j, 16)] * pv)
                    d_e = dstg[i]
                    a16 = (d_e >> 4) << 4
                    row = a16 >> 7
                    col = a16 & 127
                    win = den_tile[row, pl.ds(col, 16)]
                    hit = iota16 == (d_e - a16)
                    den_tile[row, pl.ds(col, 16)] = (
                        win + jnp.where(hit, pv, 0.0))
                return c2

            lax.fori_loop(0, K // 16, edge_grp, 0)
            pltpu.async_copy(f2_rows.at[pl.ds(sl, K), :],
                             accum.at[dst2.at[b & 1]], sem_sc, add=True)

        return carry

    lax.fori_loop(0, nb_w + 1, step, 0)

    def dsc_tail(i2, c3):
        pltpu.make_async_copy(dumz_hbm, dumv, sem_sc).wait()
        return c3

    lax.fori_loop(0, 2, dsc_tail, 0)
    fsl = lax.rem(nb_w, 3) * 16
    pltpu.make_async_copy(
        idx_hbm.at[jnp.minimum(CROWS * (w + nb_w * NW) + iota16, cmax)],
        cbuf_3.at[pl.ds(fsl, 16), :], sem_pre).wait()

    plsc.subcore_barrier()
    pltpu.sync_copy(accum.at[pl.ds(r0, ROWS_PT), :],
                    out_hbm.at[c, pl.ds(r0, ROWS_PT), :])

    def den_wb(ch, c2):
        pltpu.async_copy(den_tile.at[pl.ds(ch * 16, 16), :],
                         den_hbm.at[w * DEN_ROWS + ch * 16 + iota16],
                         sem).wait()
        return c2

    lax.fori_loop(0, DEN_ROWS // 16, den_wb, 0)


_edge_pass = functools.partial(
    pl.kernel,
    out_type=[
        jax.ShapeDtypeStruct((NCORE, N_PAD, DIM), jnp.float32),
        jax.ShapeDtypeStruct((NW * DEN_ROWS, DIM), jnp.float32),
    ],
    mesh=plsc.VectorSubcoreMesh(core_axis_name="c", subcore_axis_name="s"),
    scratch_types=[
        pltpu.VMEM((48, 128), jnp.float32),
        pltpu.VMEM((K,), jnp.int32),
        pltpu.VMEM((2, K), jnp.int32),
        pltpu.VMEM((2 * K, DIM), jnp.float32),
        pltpu.VMEM((2 * K, DIM), jnp.float32),
        pltpu.VMEM((2 * K, DIM), jnp.float32),
        pltpu.VMEM((16, 2 * DIM), jnp.float32),
        pltpu.VMEM((DEN_ROWS, DIM), jnp.float32),
        pltpu.VMEM_SHARED((N_PAD, DIM), jnp.float32),
        pltpu.SemaphoreType.DMA,
        pltpu.SemaphoreType.DMA,
        pltpu.SemaphoreType.DMA,
    ],
)(_edge_body)


def _den_body(d_ref, o_ref):
    o_ref[...] = jnp.sum(d_ref[...], axis=0)[:, None]


def _den_reduce(denp):
    blk = 1024
    grid = N_PAD // blk
    return pl.pallas_call(
        _den_body,
        grid=(grid,),
        in_specs=[pl.BlockSpec((NW, blk), lambda i: (0, i))],
        out_specs=pl.BlockSpec((blk, 1), lambda i: (i, 0)),
        out_shape=jax.ShapeDtypeStruct((N_PAD, 1), jnp.float32),
    )(denp)


def _final_body(f_ref, acc_ref, den_ref, w1_ref, b1_ref, w2_ref, b2_ref,
                delta_ref, o_ref):
    num = acc_ref[0] + acc_ref[1]
    den = den_ref[...]
    h_neigh = num / (den + 1e-9)
    f = f_ref[...]
    o = (jnp.dot(f + h_neigh, w1_ref[...], preferred_element_type=jnp.float32)
         + jnp.dot(f * h_neigh, w2_ref[...], preferred_element_type=jnp.float32)
         + b1_ref[...] + b2_ref[...])
    o = jnp.where(o >= 0.0, o, 0.2 * o)
    o_ref[...] = o + delta_ref[0, 0]


def _final(features, acc, den, w1, b1, w2, b2, delta):
    blk = 1000
    grid = N_NODES // blk
    return pl.pallas_call(
        _final_body,
        grid=(grid,),
        in_specs=[
            pl.BlockSpec((blk, DIM), lambda i: (i, 0)),
            pl.BlockSpec((NCORE, blk, DIM), lambda i: (0, i, 0)),
            pl.BlockSpec((blk, 1), lambda i: (i, 0)),
            pl.BlockSpec((DIM, DIM), lambda i: (0, 0)),
            pl.BlockSpec((1, DIM), lambda i: (0, 0)),
            pl.BlockSpec((DIM, DIM), lambda i: (0, 0)),
            pl.BlockSpec((1, DIM), lambda i: (0, 0)),
            pl.BlockSpec(memory_space=pltpu.SMEM),
        ],
        out_specs=pl.BlockSpec((blk, DIM), lambda i: (i, 0)),
        out_shape=jax.ShapeDtypeStruct((N_NODES, DIM), jnp.float32),
    )(features, acc, den, w1, b1, w2, b2, delta)


def kernel(indices, features, e_r, num_nodes, W1_w, W1_b, W2_w, W2_b,
           Watt_w, Watt_b):
    f32 = jnp.float32
    src = indices[0].astype(jnp.int32)
    dst = indices[1].astype(jnp.int32)
    features = features.astype(f32)
    e_r = e_r.astype(f32)

    s_tab, d_tab = _make_tables(features, Watt_w.astype(f32),
                                Watt_b.astype(f32).reshape(1, DIM))
    zeros = jnp.zeros((N_PAD, DIM), f32)
    idx_row = jnp.concatenate(
        [src.reshape(NB, K), dst.reshape(NB, K),
         jnp.zeros((NB, 128 - 2 * K), jnp.int32)], axis=1).astype(f32)
    er_rows = e_r.reshape(NB, K * D_EDGE // 128, 128)
    comb = jnp.concatenate([idx_row[:, None, :], er_rows],
                           axis=1).reshape(-1, 128)
    dumz = jnp.zeros((16, 2 * DIM), f32)
    acc, denp = _edge_pass(comb, s_tab, d_tab, features, zeros, dumz)
    den = _den_reduce(denp.reshape(NW, N_PAD))

    delta = jnp.asarray(num_nodes - features.shape[0], f32).reshape(1, 1)
    return _final(features, acc, den, W1_w.astype(f32),
                  W1_b.astype(f32).reshape(1, DIM), W2_w.astype(f32),
                  W2_b.astype(f32).reshape(1, DIM), delta)
